# baseline (speedup 1.0000x reference)
import functools

import jax
import jax.numpy as jnp
from jax import lax
from jax.experimental import pallas as pl
from jax.experimental.pallas import tpu as pltpu
from jax.experimental.pallas import tpu_sc as plsc

N = 10000
NP = 10240
E = 320000
E_PAD = 327680
CHUNK = 128
NH = 4
HID = 128
SMOO = 0.5

_BLK = 1024
_RPT = NP // 16
_NCH = E_PAD // 16 // CHUNK


def _lk(v, slope=0.01):
    return jnp.where(v > 0, v, slope * v)


def _k1_body(x_ref, W1_ref, b1_ref, W2_ref, b2_ref, W3_ref, b3_ref, Watt_ref,
             S_ref):
    f32 = jnp.float32
    h = _lk(jnp.dot(x_ref[...], W1_ref[...], preferred_element_type=f32)
            + b1_ref[...])
    h = _lk(jnp.dot(h, W2_ref[...], preferred_element_type=f32) + b2_ref[...])
    h = _lk(jnp.dot(h, W3_ref[...], preferred_element_type=f32) + b3_ref[...])
    for q in range(NH):
        S_ref[q] = jnp.dot(h, Watt_ref[q], preferred_element_type=f32)


def _run_k1(xp, W1, b1, W2, b2, W3, b3, Watt):
    grid = (NP // _BLK,)
    full = lambda shape: pl.BlockSpec(shape, lambda i: (0,) * len(shape))
    return pl.pallas_call(
        _k1_body,
        grid=grid,
        in_specs=[
            pl.BlockSpec((_BLK, 128), lambda i: (i, 0)),
            full((128, 512)), full((1, 512)),
            full((512, 512)), full((1, 512)),
            full((512, HID)), full((1, HID)),
            full((NH, HID, HID)),
        ],
        out_specs=pl.BlockSpec((NH, _BLK, HID), lambda i: (0, i, 0)),
        out_shape=jax.ShapeDtypeStruct((NH, NP, HID), jnp.float32),
    )(xp, W1, b1.reshape(1, 512), W2, b2.reshape(1, 512), W3,
      b3.reshape(1, HID), Watt)


def _k2_body(S_ref, C1_ref, C2_ref, C3_ref, C4_ref, aatt_ref, Wg_ref, sup_ref):
    f32 = jnp.float32
    Cs = [C1_ref, C2_ref, C3_ref, C4_ref]
    outs = []
    for i in range(NH):
        s_i = S_ref[i]
        a1 = aatt_ref[i, 0:HID, :]
        a2 = aatt_ref[i, HID:2 * HID, :]
        sdot = jnp.dot(s_i, a1, preferred_element_type=f32)
        es = []
        for C in Cs:
            e = sdot + jnp.dot(C[i], a2, preferred_element_type=f32)
            es.append(jnp.where(e > 0, e, 0.2 * e))
        m = jnp.maximum(jnp.maximum(es[0], es[1]), jnp.maximum(es[2], es[3]))
        ex = [jnp.exp(e - m) for e in es]
        tot = ex[0] + ex[1] + ex[2] + ex[3]
        o = (ex[0] / tot) * Cs[0][i]
        for j in range(1, 4):
            o = o + (ex[j] / tot) * Cs[j][i]
        outs.append(jnp.maximum(o, 0.0))
    xcat = jnp.concatenate(outs, axis=1)
    sup_ref[...] = jnp.dot(xcat, Wg_ref[...], preferred_element_type=f32)


def _run_k2(S, C1, C2, C3, C4, aatt, Wg):
    grid = (NP // _BLK,)
    cspec = pl.BlockSpec((NH, _BLK, HID), lambda i: (0, i, 0))
    full = lambda shape: pl.BlockSpec(shape, lambda i: (0,) * len(shape))
    return pl.pallas_call(
        _k2_body,
        grid=grid,
        in_specs=[cspec, cspec, cspec, cspec, cspec,
                  full((NH, 2 * HID, 1)), full((NH * HID, HID))],
        out_specs=pl.BlockSpec((_BLK, HID), lambda i: (i, 0)),
        out_shape=jax.ShapeDtypeStruct((NP, HID), jnp.float32),
    )(S, C1, C2, C3, C4, aatt, Wg)


def _k3_body(sup_ref, ps_ref, bg_ref, Wf_ref, bf_ref, out_ref):
    f32 = jnp.float32
    xg = _lk((SMOO * sup_ref[...] + ps_ref[...]) / (1.0 + SMOO) + bg_ref[...])
    out_ref[...] = jnp.dot(xg, Wf_ref[...], preferred_element_type=f32) \
        + bf_ref[...]


def _run_k3(sup, ps, bg, Wf, bf, nclass):
    grid = (NP // _BLK,)
    rspec = pl.BlockSpec((_BLK, HID), lambda i: (i, 0))
    full = lambda shape: pl.BlockSpec(shape, lambda i: (0,) * len(shape))
    return pl.pallas_call(
        _k3_body,
        grid=grid,
        in_specs=[rspec, rspec,
                  full((1, HID)), full((HID, nclass)), full((1, nclass))],
        out_specs=pl.BlockSpec((_BLK, nclass), lambda i: (i, 0)),
        out_shape=jax.ShapeDtypeStruct((NP, nclass), jnp.float32),
    )(sup, ps, bg.reshape(1, HID), Wf, bf.reshape(1, nclass))


@functools.cache
def _sc_mesh():
    return plsc.VectorSubcoreMesh(core_axis_name="c", subcore_axis_name="s")


def _sweep(table_ref, src2d_hbm, srow0, dst2d_hbm, drow0, acc, rowsA, rowsB,
           sidx8, didx8, semA, semB, semC, semD):
    @pl.loop(0, _NCH // 8)
    def _(g):
        j0 = g * 8
        pltpu.sync_copy(src2d_hbm.at[pl.ds(srow0 + j0, 8)], sidx8)
        pltpu.sync_copy(dst2d_hbm.at[pl.ds(drow0 + j0, 8)], didx8)
        bufs = [rowsA.at[pl.ds(0, 64)], rowsA.at[pl.ds(64, 64)],
                rowsB.at[pl.ds(0, 64)], rowsB.at[pl.ds(64, 64)]]
        sems = [semA, semB, semC, semD]
        for b in range(4):
            pltpu.async_copy(
                table_ref.at[sidx8.at[b // 2, pl.ds((b % 2) * 64, 64)]],
                bufs[b], sems[b])
        for k in range(16):
            pltpu.make_async_copy(
                table_ref.at[sidx8.at[k // 2, pl.ds((k % 2) * 64, 64)]],
                bufs[k % 4], sems[k % 4]).wait()
            pltpu.sync_copy(bufs[k % 4],
                            acc.at[didx8.at[k // 2, pl.ds((k % 2) * 64, 64)]],
                            add=True)
            if k + 4 < 16:
                kk = k + 4
                pltpu.async_copy(
                    table_ref.at[sidx8.at[kk // 2, pl.ds((kk % 2) * 64, 64)]],
                    bufs[k % 4], sems[k % 4])


def _zero_acc(zeros_hbm, zbuf, acc, dr0):
    pltpu.sync_copy(zeros_hbm, zbuf)
    for t in range(_RPT // 128):
        pltpu.sync_copy(zbuf, acc.at[pl.ds(dr0 + t * 128, 128)])


def _chain_body(s_hbm, srcq_hbm, dst2d_hbm, ones_hbm, zeros_hbm,
                c1_hbm, c2_hbm, c3_hbm, c4_hbm, inv_hbm,
                acc, rowsA, rowsB, sidx8, didx8, semA, semB, semC, semD):
    cid = lax.axis_index("c")
    sid = lax.axis_index("s")
    dr0 = sid * _RPT
    drow0 = sid * _NCH
    ir0 = sid * (_RPT // 8)

    pltpu.sync_copy(ones_hbm, rowsA)
    _zero_acc(zeros_hbm, rowsB, acc, dr0)
    plsc.subcore_barrier()

    @pl.loop(0, _NCH // 8)
    def _(g):
        pltpu.sync_copy(dst2d_hbm.at[pl.ds(drow0 + g * 8, 8)], didx8)
        for k in range(8):
            pltpu.sync_copy(rowsA, acc.at[didx8.at[k]], add=True)

    plsc.subcore_barrier()

    @pl.loop(0, _RPT // 128)
    def _(t):
        pltpu.sync_copy(acc.at[pl.ds(dr0 + t * 128, 128)], rowsA)

        @pl.loop(0, 128)
        def _(r):
            g = t * 128 + r
            rowsB[g // 8, pl.ds((g % 8) * 16, 16)] = \
                1.0 / jnp.maximum(rowsA[r, pl.ds(0, 16)], 1.0)

    pltpu.sync_copy(rowsB.at[pl.ds(0, _RPT // 8)],
                    inv_hbm.at[cid, pl.ds(ir0, _RPT // 8)])

    outs = [c1_hbm, c2_hbm, c3_hbm, c4_hbm]
    for ql in range(2):
        q = cid * 2 + ql
        srow0 = q * (E_PAD // CHUNK) + sid * _NCH
        tables = [s_hbm, c1_hbm, c2_hbm, c3_hbm]
        for step in range(4):
            _zero_acc(zeros_hbm, rowsB, acc, dr0)
            plsc.subcore_barrier()
            _sweep(tables[step], srcq_hbm, srow0, dst2d_hbm, drow0, acc,
                   rowsA, rowsB, sidx8, didx8, semA, semB, semC, semD)
            plsc.subcore_barrier()

            pltpu.sync_copy(inv_hbm.at[cid, pl.ds(ir0, _RPT // 8)],
                            rowsB.at[pl.ds(0, _RPT // 8)])

            @pl.loop(0, _RPT // 128)
            def _(t):
                pltpu.sync_copy(acc.at[pl.ds(dr0 + t * 128, 128)], rowsA)

                @pl.loop(0, 128)
                def _(r):
                    g = t * 128 + r
                    s = rowsB[g // 8, pl.ds((g % 8) * 16, 16)]
                    for f in range(HID // 16):
                        rowsA[r, pl.ds(f * 16, 16)] = \
                            rowsA[r, pl.ds(f * 16, 16)] * s

                pltpu.sync_copy(
                    rowsA,
                    outs[step].at[pl.ds(q * NP + dr0 + t * 128, 128)])

            plsc.subcore_barrier()


def _run_chain(S, srcq2d, dst2d):
    s_flat = S.reshape(NH * NP, HID)
    ones128 = jnp.ones((CHUNK, HID), jnp.float32)
    zeros128 = jnp.zeros((CHUNK, HID), jnp.float32)
    shp = jax.ShapeDtypeStruct((NH * NP, HID), jnp.float32)
    outs = pl.kernel(
        _chain_body,
        out_type=[shp, shp, shp, shp,
                  jax.ShapeDtypeStruct((2, NP * 16 // 128, 128), jnp.float32)],
        mesh=_sc_mesh(),
        scratch_types=[
            pltpu.VMEM_SHARED((NP, HID), jnp.float32),
            pltpu.VMEM((CHUNK, HID), jnp.float32),
            pltpu.VMEM((CHUNK, HID), jnp.float32),
            pltpu.VMEM((8, CHUNK), jnp.int32),
            pltpu.VMEM((8, CHUNK), jnp.int32),
            pltpu.SemaphoreType.DMA,
            pltpu.SemaphoreType.DMA,
            pltpu.SemaphoreType.DMA,
            pltpu.SemaphoreType.DMA,
        ],
    )(s_flat, srcq2d, dst2d, ones128, zeros128)
    c1, c2, c3, c4, inv2d = outs
    rs = lambda c: c.reshape(NH, NP, HID)
    return rs(c1), rs(c2), rs(c3), rs(c4), inv2d


_HN = NP // 2
_HACC = 6016
_HRPT = _HN // 16


def _post_body(sup_hbm, src2d_hbm, dsth2d_hbm, zeros_hbm, inv2d_hbm, out_hbm,
               acc, rowsA, rowsB, sidx8, didx8, semA, semB, semC, semD):
    cid = lax.axis_index("c")
    sid = lax.axis_index("s")
    z0 = sid * (_HACC // 16)
    jrow = sid * _NCH

    pltpu.sync_copy(zeros_hbm, rowsB)
    for off, sz in ((0, 128), (128, 128), (256, 120)):
        pltpu.sync_copy(rowsB.at[pl.ds(0, sz)], acc.at[pl.ds(z0 + off, sz)])
    plsc.subcore_barrier()

    _sweep(sup_hbm, src2d_hbm, jrow, dsth2d_hbm.at[cid], jrow, acc,
           rowsA, rowsB, sidx8, didx8, semA, semB, semC, semD)
    plsc.subcore_barrier()

    pltpu.sync_copy(
        inv2d_hbm.at[0, pl.ds(cid * (_HN // 8) + sid * (_HRPT // 8),
                              _HRPT // 8)],
        rowsB.at[pl.ds(0, _HRPT // 8)])
    dr0 = sid * _HRPT
    for t, sz in ((0, 128), (1, 128), (2, 64)):
        pltpu.sync_copy(acc.at[pl.ds(dr0 + t * 128, sz)],
                        rowsA.at[pl.ds(0, sz)])

        @pl.loop(0, sz)
        def _(r):
            g = t * 128 + r
            s = rowsB[g // 8, pl.ds((g % 8) * 16, 16)]
            for f in range(HID // 16):
                rowsA[r, pl.ds(f * 16, 16)] = rowsA[r, pl.ds(f * 16, 16)] * s

        pltpu.sync_copy(rowsA.at[pl.ds(0, sz)],
                        out_hbm.at[pl.ds(cid * _HN + dr0 + t * 128, sz)])


def _run_post(sup, src2d, dsth2d, inv2d):
    zeros128 = jnp.zeros((CHUNK, HID), jnp.float32)
    return pl.kernel(
        _post_body,
        out_type=jax.ShapeDtypeStruct((NP, HID), jnp.float32),
        mesh=_sc_mesh(),
        scratch_types=[
            pltpu.VMEM_SHARED((_HACC, HID), jnp.float32),
            pltpu.VMEM((CHUNK, HID), jnp.float32),
            pltpu.VMEM((CHUNK, HID), jnp.float32),
            pltpu.VMEM((8, CHUNK), jnp.int32),
            pltpu.VMEM((8, CHUNK), jnp.int32),
            pltpu.SemaphoreType.DMA,
            pltpu.SemaphoreType.DMA,
            pltpu.SemaphoreType.DMA,
            pltpu.SemaphoreType.DMA,
        ],
    )(sup, src2d, dsth2d, zeros128, inv2d)


def kernel(x, adj, W1, b1, W2, b2, W3, b3, Watt, aatt, Wg, bg, Wf, bf):
    nclass = Wf.shape[1]
    src = adj[0]
    dst = adj[1]
    pad = E_PAD - E
    srcp = jnp.concatenate([src, jnp.zeros((pad,), jnp.int32)])
    dstp = jnp.concatenate([dst, jnp.full((pad,), N, jnp.int32)])
    srcq2d = (srcp[None, :]
              + NP * jnp.arange(NH, dtype=jnp.int32)[:, None]
              ).reshape(NH * E_PAD // CHUNK, CHUNK)
    dst2d = dstp.reshape(E_PAD // CHUNK, CHUNK)
    half = jnp.arange(2, dtype=jnp.int32)[:, None] * _HN
    inh = (dstp[None, :] >= half) & (dstp[None, :] < half + _HN)
    dsth2d = jnp.where(inh, dstp[None, :] - half, _HN).astype(
        jnp.int32).reshape(2, E_PAD // CHUNK, CHUNK)
    src2d = srcp.reshape(E_PAD // CHUNK, CHUNK)
    xp = jnp.pad(x, ((0, NP - N), (0, 0)))

    S = _run_k1(xp, W1, b1, W2, b2, W3, b3, Watt)
    C1, C2, C3, C4, inv2d = _run_chain(S, srcq2d, dst2d)
    sup = _run_k2(S, C1, C2, C3, C4, aatt, Wg)
    ps = _run_post(sup, src2d, dsth2d, inv2d)
    return _run_k3(sup, ps, bg, Wf, bf, nclass)[:N]


# speedup vs baseline: 1.0572x; 1.0572x over previous
import functools

import jax
import jax.numpy as jnp
from jax import lax
from jax.experimental import pallas as pl
from jax.experimental.pallas import tpu as pltpu
from jax.experimental.pallas import tpu_sc as plsc

N = 10000
NP = 10240
E = 320000
E_PAD = 327680
CHUNK = 128
NH = 4
HID = 128
SMOO = 0.5

_BLK = 1024
_RPT = NP // 16
_NCH = E_PAD // 16 // CHUNK


def _lk(v, slope=0.01):
    return jnp.where(v > 0, v, slope * v)


def _k1_body(x_ref, W1_ref, b1_ref, W2_ref, b2_ref, W3_ref, b3_ref, Watt_ref,
             S_ref):
    f32 = jnp.float32
    h = _lk(jnp.dot(x_ref[...], W1_ref[...], preferred_element_type=f32)
            + b1_ref[...])
    h = _lk(jnp.dot(h, W2_ref[...], preferred_element_type=f32) + b2_ref[...])
    h = _lk(jnp.dot(h, W3_ref[...], preferred_element_type=f32) + b3_ref[...])
    for q in range(NH):
        S_ref[q] = jnp.dot(h, Watt_ref[q], preferred_element_type=f32)


def _run_k1(xp, W1, b1, W2, b2, W3, b3, Watt):
    grid = (NP // _BLK,)
    full = lambda shape: pl.BlockSpec(shape, lambda i: (0,) * len(shape))
    return pl.pallas_call(
        _k1_body,
        grid=grid,
        in_specs=[
            pl.BlockSpec((_BLK, 128), lambda i: (i, 0)),
            full((128, 512)), full((1, 512)),
            full((512, 512)), full((1, 512)),
            full((512, HID)), full((1, HID)),
            full((NH, HID, HID)),
        ],
        out_specs=pl.BlockSpec((NH, _BLK, HID), lambda i: (0, i, 0)),
        out_shape=jax.ShapeDtypeStruct((NH, NP, HID), jnp.float32),
    )(xp, W1, b1.reshape(1, 512), W2, b2.reshape(1, 512), W3,
      b3.reshape(1, HID), Watt)


def _k2_body(S_ref, C1_ref, C2_ref, C3_ref, C4_ref, aatt_ref, Wg_ref, sup_ref):
    f32 = jnp.float32
    Cs = [C1_ref, C2_ref, C3_ref, C4_ref]
    outs = []
    for i in range(NH):
        s_i = S_ref[i]
        a1 = aatt_ref[i, 0:HID, :]
        a2 = aatt_ref[i, HID:2 * HID, :]
        sdot = jnp.dot(s_i, a1, preferred_element_type=f32)
        es = []
        for C in Cs:
            e = sdot + jnp.dot(C[i], a2, preferred_element_type=f32)
            es.append(jnp.where(e > 0, e, 0.2 * e))
        m = jnp.maximum(jnp.maximum(es[0], es[1]), jnp.maximum(es[2], es[3]))
        ex = [jnp.exp(e - m) for e in es]
        tot = ex[0] + ex[1] + ex[2] + ex[3]
        o = (ex[0] / tot) * Cs[0][i]
        for j in range(1, 4):
            o = o + (ex[j] / tot) * Cs[j][i]
        outs.append(jnp.maximum(o, 0.0))
    xcat = jnp.concatenate(outs, axis=1)
    sup_ref[...] = jnp.dot(xcat, Wg_ref[...], preferred_element_type=f32)


def _run_k2(S, C1, C2, C3, C4, aatt, Wg):
    grid = (NP // _BLK,)
    cspec = pl.BlockSpec((NH, _BLK, HID), lambda i: (0, i, 0))
    full = lambda shape: pl.BlockSpec(shape, lambda i: (0,) * len(shape))
    return pl.pallas_call(
        _k2_body,
        grid=grid,
        in_specs=[cspec, cspec, cspec, cspec, cspec,
                  full((NH, 2 * HID, 1)), full((NH * HID, HID))],
        out_specs=pl.BlockSpec((_BLK, HID), lambda i: (i, 0)),
        out_shape=jax.ShapeDtypeStruct((NP, HID), jnp.float32),
    )(S, C1, C2, C3, C4, aatt, Wg)


def _k3_body(sup_ref, pp_ref, inv_ref, bg_ref, Wf_ref, bf_ref, out_ref):
    f32 = jnp.float32
    psum = (pp_ref[0] + pp_ref[1]) * inv_ref[:, 0:1]
    xg = _lk((SMOO * sup_ref[...] + psum) / (1.0 + SMOO) + bg_ref[...])
    out_ref[...] = jnp.dot(xg, Wf_ref[...], preferred_element_type=f32) \
        + bf_ref[...]


def _run_k3(sup, pp, inv16, bg, Wf, bf, nclass):
    grid = (NP // _BLK,)
    rspec = pl.BlockSpec((_BLK, HID), lambda i: (i, 0))
    full = lambda shape: pl.BlockSpec(shape, lambda i: (0,) * len(shape))
    return pl.pallas_call(
        _k3_body,
        grid=grid,
        in_specs=[rspec,
                  pl.BlockSpec((2, _BLK, HID), lambda i: (0, i, 0)),
                  pl.BlockSpec((_BLK, 16), lambda i: (i, 0)),
                  full((1, HID)), full((HID, nclass)), full((1, nclass))],
        out_specs=pl.BlockSpec((_BLK, nclass), lambda i: (i, 0)),
        out_shape=jax.ShapeDtypeStruct((NP, nclass), jnp.float32),
    )(sup, pp, inv16, bg.reshape(1, HID), Wf, bf.reshape(1, nclass))


@functools.cache
def _sc_mesh():
    return plsc.VectorSubcoreMesh(core_axis_name="c", subcore_axis_name="s")


def _sweep(table_ref, src2d_hbm, srow0, dst2d_hbm, drow0, acc, rowsA, rowsB,
           sidx8, didx8, semA, semB, semC, semD, ngroups=_NCH // 8):
    @pl.loop(0, ngroups)
    def _(g):
        j0 = g * 8
        pltpu.sync_copy(src2d_hbm.at[pl.ds(srow0 + j0, 8)], sidx8)
        pltpu.sync_copy(dst2d_hbm.at[pl.ds(drow0 + j0, 8)], didx8)
        bufs = [rowsA.at[pl.ds(0, 64)], rowsA.at[pl.ds(64, 64)],
                rowsB.at[pl.ds(0, 64)], rowsB.at[pl.ds(64, 64)]]
        sems = [semA, semB, semC, semD]
        for b in range(4):
            pltpu.async_copy(
                table_ref.at[sidx8.at[b // 2, pl.ds((b % 2) * 64, 64)]],
                bufs[b], sems[b])
        for k in range(16):
            pltpu.make_async_copy(
                table_ref.at[sidx8.at[k // 2, pl.ds((k % 2) * 64, 64)]],
                bufs[k % 4], sems[k % 4]).wait()
            pltpu.sync_copy(bufs[k % 4],
                            acc.at[didx8.at[k // 2, pl.ds((k % 2) * 64, 64)]],
                            add=True)
            if k + 4 < 16:
                kk = k + 4
                pltpu.async_copy(
                    table_ref.at[sidx8.at[kk // 2, pl.ds((kk % 2) * 64, 64)]],
                    bufs[k % 4], sems[k % 4])


def _zero_acc(zeros_hbm, zbuf, acc, dr0):
    pltpu.sync_copy(zeros_hbm, zbuf)
    for t in range(_RPT // 128):
        pltpu.sync_copy(zbuf, acc.at[pl.ds(dr0 + t * 128, 128)])


def _chain_body(s_hbm, srcq_hbm, dst2d_hbm, ones_hbm, zeros_hbm,
                c1_hbm, c2_hbm, c3_hbm, c4_hbm, inv_hbm,
                acc, rowsA, rowsB, sidx8, didx8, semA, semB, semC, semD):
    cid = lax.axis_index("c")
    sid = lax.axis_index("s")
    dr0 = sid * _RPT
    drow0 = sid * _NCH
    ir0 = sid * (_RPT // 8)

    pltpu.sync_copy(ones_hbm, rowsA)
    _zero_acc(zeros_hbm, rowsB, acc, dr0)
    plsc.subcore_barrier()

    @pl.loop(0, _NCH // 8)
    def _(g):
        pltpu.sync_copy(dst2d_hbm.at[pl.ds(drow0 + g * 8, 8)], didx8)
        for k in range(8):
            pltpu.sync_copy(rowsA, acc.at[didx8.at[k]], add=True)

    plsc.subcore_barrier()

    @pl.loop(0, _RPT // 128)
    def _(t):
        pltpu.sync_copy(acc.at[pl.ds(dr0 + t * 128, 128)], rowsA)

        @pl.loop(0, 128)
        def _(r):
            g = t * 128 + r
            rowsB[g // 8, pl.ds((g % 8) * 16, 16)] = \
                1.0 / jnp.maximum(rowsA[r, pl.ds(0, 16)], 1.0)

    pltpu.sync_copy(rowsB.at[pl.ds(0, _RPT // 8)],
                    inv_hbm.at[cid, pl.ds(ir0, _RPT // 8)])

    outs = [c1_hbm, c2_hbm, c3_hbm, c4_hbm]
    for ql in range(2):
        q = cid * 2 + ql
        srow0 = q * (E_PAD // CHUNK) + sid * _NCH
        tables = [s_hbm, c1_hbm, c2_hbm, c3_hbm]
        for step in range(4):
            _zero_acc(zeros_hbm, rowsB, acc, dr0)
            plsc.subcore_barrier()
            _sweep(tables[step], srcq_hbm, srow0, dst2d_hbm, drow0, acc,
                   rowsA, rowsB, sidx8, didx8, semA, semB, semC, semD)
            plsc.subcore_barrier()

            pltpu.sync_copy(inv_hbm.at[cid, pl.ds(ir0, _RPT // 8)],
                            rowsB.at[pl.ds(0, _RPT // 8)])

            @pl.loop(0, _RPT // 128)
            def _(t):
                pltpu.sync_copy(acc.at[pl.ds(dr0 + t * 128, 128)], rowsA)

                @pl.loop(0, 128)
                def _(r):
                    g = t * 128 + r
                    s = rowsB[g // 8, pl.ds((g % 8) * 16, 16)]
                    for f in range(HID // 16):
                        rowsA[r, pl.ds(f * 16, 16)] = \
                            rowsA[r, pl.ds(f * 16, 16)] * s

                pltpu.sync_copy(
                    rowsA,
                    outs[step].at[pl.ds(q * NP + dr0 + t * 128, 128)])

            plsc.subcore_barrier()


def _run_chain(S, srcq2d, dst2d):
    s_flat = S.reshape(NH * NP, HID)
    ones128 = jnp.ones((CHUNK, HID), jnp.float32)
    zeros128 = jnp.zeros((CHUNK, HID), jnp.float32)
    shp = jax.ShapeDtypeStruct((NH * NP, HID), jnp.float32)
    outs = pl.kernel(
        _chain_body,
        out_type=[shp, shp, shp, shp,
                  jax.ShapeDtypeStruct((2, NP * 16 // 128, 128), jnp.float32)],
        mesh=_sc_mesh(),
        scratch_types=[
            pltpu.VMEM_SHARED((NP, HID), jnp.float32),
            pltpu.VMEM((CHUNK, HID), jnp.float32),
            pltpu.VMEM((CHUNK, HID), jnp.float32),
            pltpu.VMEM((8, CHUNK), jnp.int32),
            pltpu.VMEM((8, CHUNK), jnp.int32),
            pltpu.SemaphoreType.DMA,
            pltpu.SemaphoreType.DMA,
            pltpu.SemaphoreType.DMA,
            pltpu.SemaphoreType.DMA,
        ],
    )(s_flat, srcq2d, dst2d, ones128, zeros128)
    c1, c2, c3, c4, inv2d = outs
    rs = lambda c: c.reshape(NH, NP, HID)
    return rs(c1), rs(c2), rs(c3), rs(c4), inv2d


_HN = NP // 2
_HACC = 6016
_HRPT = _HN // 16


def _post_body(sup_hbm, src2d_hbm, dst2d_hbm, zeros_hbm, out_hbm,
               acc, rowsA, rowsB, sidx8, didx8, semA, semB, semC, semD):
    cid = lax.axis_index("c")
    sid = lax.axis_index("s")
    dr0 = sid * _RPT
    wrow0 = (cid * 16 + sid) * (_NCH // 2)

    _zero_acc(zeros_hbm, rowsB, acc, dr0)
    plsc.subcore_barrier()
    _sweep(sup_hbm, src2d_hbm, wrow0, dst2d_hbm, wrow0, acc,
           rowsA, rowsB, sidx8, didx8, semA, semB, semC, semD,
           ngroups=_NCH // 16)
    plsc.subcore_barrier()
    pltpu.sync_copy(acc.at[pl.ds(dr0, _RPT)],
                    out_hbm.at[cid, pl.ds(dr0, _RPT)])


def _run_post(sup, src2d, dst2d):
    zeros128 = jnp.zeros((CHUNK, HID), jnp.float32)
    return pl.kernel(
        _post_body,
        out_type=jax.ShapeDtypeStruct((2, NP, HID), jnp.float32),
        mesh=_sc_mesh(),
        scratch_types=[
            pltpu.VMEM_SHARED((NP, HID), jnp.float32),
            pltpu.VMEM((CHUNK, HID), jnp.float32),
            pltpu.VMEM((CHUNK, HID), jnp.float32),
            pltpu.VMEM((8, CHUNK), jnp.int32),
            pltpu.VMEM((8, CHUNK), jnp.int32),
            pltpu.SemaphoreType.DMA,
            pltpu.SemaphoreType.DMA,
            pltpu.SemaphoreType.DMA,
            pltpu.SemaphoreType.DMA,
        ],
    )(sup, src2d, dst2d, zeros128)


def kernel(x, adj, W1, b1, W2, b2, W3, b3, Watt, aatt, Wg, bg, Wf, bf):
    nclass = Wf.shape[1]
    src = adj[0]
    dst = adj[1]
    pad = E_PAD - E
    srcp = jnp.concatenate([src, jnp.zeros((pad,), jnp.int32)])
    dstp = jnp.concatenate([dst, jnp.full((pad,), N, jnp.int32)])
    srcq2d = (srcp[None, :]
              + NP * jnp.arange(NH, dtype=jnp.int32)[:, None]
              ).reshape(NH * E_PAD // CHUNK, CHUNK)
    dst2d = dstp.reshape(E_PAD // CHUNK, CHUNK)
    src2d = srcp.reshape(E_PAD // CHUNK, CHUNK)
    xp = jnp.pad(x, ((0, NP - N), (0, 0)))

    S = _run_k1(xp, W1, b1, W2, b2, W3, b3, Watt)
    C1, C2, C3, C4, inv2d = _run_chain(S, srcq2d, dst2d)
    sup = _run_k2(S, C1, C2, C3, C4, aatt, Wg)
    pp = _run_post(sup, src2d, dst2d)
    inv16 = inv2d[0].reshape(NP, 16)
    return _run_k3(sup, pp, inv16, bg, Wf, bf, nclass)[:N]
